# baseline (speedup 1.0000x reference)
import functools

import jax
import jax.numpy as jnp
from jax.experimental import pallas as pl
from jax.experimental.pallas import tpu as pltpu


def _router_block_kernel(x_ref, w1_ref, b1_ref, w2_ref, b2_ref, out_ref):
    x_bf16 = x_ref[:].astype(jnp.bfloat16)
    h = jnp.dot(x_bf16, w1_ref[:], preferred_element_type=jnp.float32)
    h = jnp.maximum(h + b1_ref[:], 0.0)
    logits = jnp.dot(h.astype(jnp.bfloat16), w2_ref[:],
                     preferred_element_type=jnp.float32)
    logits = logits + b2_ref[:]

    z = logits - jnp.max(logits, axis=1, keepdims=True)
    e = jnp.exp(z)
    p = e / jnp.sum(e, axis=1, keepdims=True)

    n_exp = p.shape[1]
    col = jax.lax.broadcasted_iota(jnp.int32, p.shape, 1)
    m1 = jnp.max(p, axis=1, keepdims=True)
    i1 = jnp.min(jnp.where(p >= m1, col, n_exp), axis=1, keepdims=True)
    p_rest = jnp.where(col == i1, -jnp.inf, p)
    m2 = jnp.max(p_rest, axis=1, keepdims=True)
    i2 = jnp.min(jnp.where(p_rest >= m2, col, n_exp), axis=1, keepdims=True)

    mask = (col == i1) | (col == i2)
    out_ref[:] = jnp.where(mask, p, 0.0) / (m1 + m2 + 1e-8)


@functools.partial(jax.jit, static_argnames=())
def kernel(x, W1, b1, W2, b2):
    n_tokens, d_in = x.shape
    d_hidden = W1.shape[1]
    n_experts = W2.shape[1]
    bm = 512
    grid = (n_tokens // bm,)

    b1_2d = b1.reshape(1, d_hidden)
    b2_2d = b2.reshape(1, n_experts)
    w1_bf16 = W1.astype(jnp.bfloat16)
    w2_bf16 = W2.astype(jnp.bfloat16)

    return pl.pallas_call(
        _router_block_kernel,
        grid=grid,
        in_specs=[
            pl.BlockSpec((bm, d_in), lambda i: (i, 0)),
            pl.BlockSpec((d_in, d_hidden), lambda i: (0, 0)),
            pl.BlockSpec((1, d_hidden), lambda i: (0, 0)),
            pl.BlockSpec((d_hidden, n_experts), lambda i: (0, 0)),
            pl.BlockSpec((1, n_experts), lambda i: (0, 0)),
        ],
        out_specs=pl.BlockSpec((bm, n_experts), lambda i: (i, 0)),
        out_shape=jax.ShapeDtypeStruct((n_tokens, n_experts), jnp.float32),
        compiler_params=pltpu.CompilerParams(
            dimension_semantics=("parallel",),
        ),
    )(x, w1_bf16, b1_2d, w2_bf16, b2_2d)


# speedup vs baseline: 1.0355x; 1.0355x over previous
import functools

import jax
import jax.numpy as jnp
from jax.experimental import pallas as pl
from jax.experimental.pallas import tpu as pltpu


def _router_block_kernel(x_ref, w1_ref, b1_ref, w2_ref, b2_ref, out_ref,
                         w1bf_scr, logits_scr):
    i = pl.program_id(0)
    nsteps = pl.num_programs(0)

    @pl.when(i == 0)
    def _():
        w1bf_scr[:] = w1_ref[:].astype(jnp.bfloat16)

    @pl.when(i < nsteps - 1)
    def _():
        x_bf16 = x_ref[:].astype(jnp.bfloat16)
        h = jnp.dot(x_bf16, w1bf_scr[:], preferred_element_type=jnp.float32)
        h = jnp.maximum(h + b1_ref[:], 0.0)
        logits = jnp.dot(h.astype(jnp.bfloat16),
                         w2_ref[:].astype(jnp.bfloat16),
                         preferred_element_type=jnp.float32)
        logits_scr[i % 2] = logits + b2_ref[:]

    @pl.when(i > 0)
    def _():
        logits = logits_scr[(i + 1) % 2]
        z = logits - jnp.max(logits, axis=1, keepdims=True)
        e = jnp.exp(z)
        p = e / jnp.sum(e, axis=1, keepdims=True)

        n_exp = p.shape[1]
        col = jax.lax.broadcasted_iota(jnp.int32, p.shape, 1)
        m1 = jnp.max(p, axis=1, keepdims=True)
        i1 = jnp.min(jnp.where(p >= m1, col, n_exp), axis=1, keepdims=True)
        p_rest = jnp.where(col == i1, -jnp.inf, p)
        m2 = jnp.max(p_rest, axis=1, keepdims=True)
        i2 = jnp.min(jnp.where(p_rest >= m2, col, n_exp), axis=1,
                     keepdims=True)

        mask = (col == i1) | (col == i2)
        out_ref[:] = jnp.where(mask, p, 0.0) / (m1 + m2 + 1e-8)


@functools.partial(jax.jit, static_argnames=())
def kernel(x, W1, b1, W2, b2):
    n_tokens, d_in = x.shape
    d_hidden = W1.shape[1]
    n_experts = W2.shape[1]
    bm = 512
    nblk = n_tokens // bm
    grid = (nblk + 1,)

    b1_2d = b1.reshape(1, d_hidden)
    b2_2d = b2.reshape(1, n_experts)

    return pl.pallas_call(
        _router_block_kernel,
        grid=grid,
        in_specs=[
            pl.BlockSpec((bm, d_in), lambda i: (jnp.minimum(i, nblk - 1), 0)),
            pl.BlockSpec((d_in, d_hidden), lambda i: (0, 0)),
            pl.BlockSpec((1, d_hidden), lambda i: (0, 0)),
            pl.BlockSpec((d_hidden, n_experts), lambda i: (0, 0)),
            pl.BlockSpec((1, n_experts), lambda i: (0, 0)),
        ],
        out_specs=pl.BlockSpec((bm, n_experts),
                               lambda i: (jnp.maximum(i - 1, 0), 0)),
        out_shape=jax.ShapeDtypeStruct((n_tokens, n_experts), jnp.float32),
        scratch_shapes=[
            pltpu.VMEM((d_in, d_hidden), jnp.bfloat16),
            pltpu.VMEM((2, bm, n_experts), jnp.float32),
        ],
        compiler_params=pltpu.CompilerParams(
            dimension_semantics=("arbitrary",),
        ),
    )(x, W1, b1_2d, W2, b2_2d)


# speedup vs baseline: 1.1153x; 1.0770x over previous
import functools

import jax
import jax.numpy as jnp
from jax.experimental import pallas as pl
from jax.experimental.pallas import tpu as pltpu


def _router_block_kernel(x_ref, w1_ref, b1_ref, w2_ref, b2_ref, out_ref,
                         w1bf_scr, logits_scr):
    i = pl.program_id(0)

    @pl.when(i == 0)
    def _():
        w1bf_scr[:] = w1_ref[:].astype(jnp.bfloat16)

    x_bf16 = x_ref[:].astype(jnp.bfloat16)
    h = jnp.dot(x_bf16, w1bf_scr[:], preferred_element_type=jnp.float32)
    h = jnp.maximum(h + b1_ref[:], 0.0)
    new_logits = jnp.dot(h.astype(jnp.bfloat16),
                         w2_ref[:].astype(jnp.bfloat16),
                         preferred_element_type=jnp.float32) + b2_ref[:]

    logits = logits_scr[(i + 1) % 2]
    z = logits - jnp.max(logits, axis=1, keepdims=True)
    e = jnp.exp(z)
    p = e / jnp.sum(e, axis=1, keepdims=True)

    n_exp = p.shape[1]
    col = jax.lax.broadcasted_iota(jnp.int32, p.shape, 1)
    m1 = jnp.max(p, axis=1, keepdims=True)
    i1 = jnp.min(jnp.where(p >= m1, col, n_exp), axis=1, keepdims=True)
    p_rest = jnp.where(col == i1, -jnp.inf, p)
    m2 = jnp.max(p_rest, axis=1, keepdims=True)
    i2 = jnp.min(jnp.where(p_rest >= m2, col, n_exp), axis=1, keepdims=True)

    mask = (col == i1) | (col == i2)
    out_ref[:] = jnp.where(mask, p, 0.0) / (m1 + m2 + 1e-8)

    logits_scr[i % 2] = new_logits


@functools.partial(jax.jit, static_argnames=())
def kernel(x, W1, b1, W2, b2):
    n_tokens, d_in = x.shape
    d_hidden = W1.shape[1]
    n_experts = W2.shape[1]
    bm = 512
    nblk = n_tokens // bm
    grid = (nblk + 1,)

    b1_2d = b1.reshape(1, d_hidden)
    b2_2d = b2.reshape(1, n_experts)

    return pl.pallas_call(
        _router_block_kernel,
        grid=grid,
        in_specs=[
            pl.BlockSpec((bm, d_in), lambda i: (jnp.minimum(i, nblk - 1), 0)),
            pl.BlockSpec((d_in, d_hidden), lambda i: (0, 0)),
            pl.BlockSpec((1, d_hidden), lambda i: (0, 0)),
            pl.BlockSpec((d_hidden, n_experts), lambda i: (0, 0)),
            pl.BlockSpec((1, n_experts), lambda i: (0, 0)),
        ],
        out_specs=pl.BlockSpec((bm, n_experts),
                               lambda i: (jnp.maximum(i - 1, 0), 0)),
        out_shape=jax.ShapeDtypeStruct((n_tokens, n_experts), jnp.float32),
        scratch_shapes=[
            pltpu.VMEM((d_in, d_hidden), jnp.bfloat16),
            pltpu.VMEM((2, bm, n_experts), jnp.float32),
        ],
        compiler_params=pltpu.CompilerParams(
            dimension_semantics=("arbitrary",),
        ),
    )(x, W1, b1_2d, W2, b2_2d)
